# baseline (speedup 1.0000x reference)
import functools

import jax
import jax.numpy as jnp
from jax.experimental import pallas as pl
from jax.experimental.pallas import tpu as pltpu

_CHUNK = 512


def _matmul_kernel(x_hbm, w_ref, o_ref, xb_ref, xwin_ref, sem):
    j = pl.program_id(0)
    m = xb_ref.shape[0]
    n_chunks = m // _CHUNK

    @pl.when(j == 0)
    def _pack_x():
        def copy_in(c, slot):
            return pltpu.make_async_copy(
                x_hbm.at[pl.ds(c * _CHUNK, _CHUNK), :],
                xwin_ref.at[slot],
                sem.at[slot],
            )

        copy_in(0, 0).start()
        for c in range(n_chunks):
            slot = c % 2
            if c + 1 < n_chunks:
                copy_in(c + 1, (c + 1) % 2).start()
            copy_in(c, slot).wait()
            xb_ref[pl.ds(c * _CHUNK, _CHUNK), :] = (
                xwin_ref[slot].astype(jnp.bfloat16))

    o_ref[...] = jax.lax.dot_general(
        xb_ref[...], w_ref[...].astype(jnp.bfloat16),
        dimension_numbers=(((1,), (1,)), ((), ())),
        preferred_element_type=jnp.float32,
    )


@functools.partial(jax.jit, static_argnames=("bn",))
def _matmul(x, W, bn=256):
    m, k = x.shape
    n, k2 = W.shape
    grid = (n // bn,)
    return pl.pallas_call(
        _matmul_kernel,
        grid=grid,
        in_specs=[
            pl.BlockSpec(memory_space=pl.ANY),
            pl.BlockSpec((bn, k2), lambda j: (j, 0)),
        ],
        out_specs=pl.BlockSpec((m, bn), lambda j: (0, j)),
        out_shape=jax.ShapeDtypeStruct((m, n), jnp.float32),
        scratch_shapes=[
            pltpu.VMEM((m, k), jnp.bfloat16),
            pltpu.VMEM((2, _CHUNK, k), jnp.float32),
            pltpu.SemaphoreType.DMA((2,)),
        ],
        compiler_params=pltpu.CompilerParams(
            vmem_limit_bytes=60 * 1024 * 1024,
        ),
    )(x, W)


def kernel(x, W):
    return _matmul(x, W)


# speedup vs baseline: 1.0015x; 1.0015x over previous
import functools

import jax
import jax.numpy as jnp
from jax.experimental import pallas as pl
from jax.experimental.pallas import tpu as pltpu

_CHUNK = 512


def _matmul_kernel(x_hbm, w_ref, o_ref, xb_ref, xwin_ref, sem):
    j = pl.program_id(0)
    m = xb_ref.shape[0]
    n_chunks = m // _CHUNK

    @pl.when(j == 0)
    def _pack_x():
        def copy_in(c, slot):
            return pltpu.make_async_copy(
                x_hbm.at[pl.ds(c * _CHUNK, _CHUNK), :],
                xwin_ref.at[slot],
                sem.at[slot],
            )

        copy_in(0, 0).start()
        for c in range(n_chunks):
            slot = c % 2
            if c + 1 < n_chunks:
                copy_in(c + 1, (c + 1) % 2).start()
            copy_in(c, slot).wait()
            xb_ref[pl.ds(c * _CHUNK, _CHUNK), :] = (
                xwin_ref[slot].astype(jnp.bfloat16))

    o_ref[...] = jax.lax.dot_general(
        xb_ref[...], w_ref[...].astype(jnp.bfloat16),
        dimension_numbers=(((1,), (1,)), ((), ())),
        preferred_element_type=jnp.float32,
    )


@functools.partial(jax.jit, static_argnames=("bn",))
def _matmul(x, W, bn=512):
    m, k = x.shape
    n, k2 = W.shape
    grid = (n // bn,)
    return pl.pallas_call(
        _matmul_kernel,
        grid=grid,
        in_specs=[
            pl.BlockSpec(memory_space=pl.ANY),
            pl.BlockSpec((bn, k2), lambda j: (j, 0)),
        ],
        out_specs=pl.BlockSpec((m, bn), lambda j: (0, j)),
        out_shape=jax.ShapeDtypeStruct((m, n), jnp.float32),
        scratch_shapes=[
            pltpu.VMEM((m, k), jnp.bfloat16),
            pltpu.VMEM((2, _CHUNK, k), jnp.float32),
            pltpu.SemaphoreType.DMA((2,)),
        ],
        compiler_params=pltpu.CompilerParams(
            vmem_limit_bytes=60 * 1024 * 1024,
        ),
    )(x, W)


def kernel(x, W):
    return _matmul(x, W)


# speedup vs baseline: 1.0020x; 1.0005x over previous
import functools

import jax
import jax.numpy as jnp
from jax.experimental import pallas as pl
from jax.experimental.pallas import tpu as pltpu

_WTILE = 256


def _dot_nt(a, b):
    return jax.lax.dot_general(
        a, b, dimension_numbers=(((1,), (1,)), ((), ())),
        preferred_element_type=jnp.float32,
    )


def _matmul_kernel(x_ref, w_hbm, o_ref, wb_ref, wwin_ref, sem):
    i = pl.program_id(0)
    n = wb_ref.shape[0]
    n_tiles = n // _WTILE

    xb = x_ref[...].astype(jnp.bfloat16)

    @pl.when(i == 0)
    def _pack_w_and_compute():
        def copy_in(t, slot):
            return pltpu.make_async_copy(
                w_hbm.at[pl.ds(t * _WTILE, _WTILE), :],
                wwin_ref.at[slot],
                sem.at[slot],
            )

        copy_in(0, 0).start()
        for t in range(n_tiles):
            slot = t % 2
            if t + 1 < n_tiles:
                copy_in(t + 1, (t + 1) % 2).start()
            copy_in(t, slot).wait()
            wtile = wwin_ref[slot].astype(jnp.bfloat16)
            wb_ref[pl.ds(t * _WTILE, _WTILE), :] = wtile
            o_ref[:, pl.ds(t * _WTILE, _WTILE)] = _dot_nt(xb, wtile)

    @pl.when(i != 0)
    def _compute():
        o_ref[...] = _dot_nt(xb, wb_ref[...])


@functools.partial(jax.jit, static_argnames=("bm",))
def _matmul(x, W, bm=512):
    m, k = x.shape
    n, k2 = W.shape
    grid = (m // bm,)
    return pl.pallas_call(
        _matmul_kernel,
        grid=grid,
        in_specs=[
            pl.BlockSpec((bm, k), lambda i: (i, 0)),
            pl.BlockSpec(memory_space=pl.ANY),
        ],
        out_specs=pl.BlockSpec((bm, n), lambda i: (i, 0)),
        out_shape=jax.ShapeDtypeStruct((m, n), jnp.float32),
        scratch_shapes=[
            pltpu.VMEM((n, k), jnp.bfloat16),
            pltpu.VMEM((2, _WTILE, k), jnp.float32),
            pltpu.SemaphoreType.DMA((2,)),
        ],
        compiler_params=pltpu.CompilerParams(
            vmem_limit_bytes=60 * 1024 * 1024,
        ),
    )(x, W)


def kernel(x, W):
    return _matmul(x, W)


# speedup vs baseline: 1.0313x; 1.0293x over previous
import functools

import jax
import jax.numpy as jnp
from jax.experimental import pallas as pl


def _matmul_kernel(x_ref, w_ref, o_ref):
    o_ref[...] = jax.lax.dot_general(
        x_ref[...], w_ref[...],
        dimension_numbers=(((1,), (1,)), ((), ())),
        preferred_element_type=jnp.float32,
    )


@functools.partial(jax.jit, static_argnames=("bn",))
def _matmul(x, W, bn=256):
    m, k = x.shape
    n, k2 = W.shape
    grid = (n // bn,)
    return pl.pallas_call(
        _matmul_kernel,
        grid=grid,
        in_specs=[
            pl.BlockSpec((m, k), lambda j: (0, 0)),
            pl.BlockSpec((bn, k2), lambda j: (j, 0)),
        ],
        out_specs=pl.BlockSpec((m, bn), lambda j: (0, j)),
        out_shape=jax.ShapeDtypeStruct((m, n), jnp.float32),
    )(x, W)


def kernel(x, W):
    return _matmul(x, W)
